# baseline (speedup 1.0000x reference)
import functools

import jax
import jax.numpy as jnp
from jax import lax
from jax.experimental import pallas as pl
from jax.experimental.pallas import tpu as pltpu
from jax.experimental.pallas import tpu_sc as plsc

VOCAB_SIZE = 50257
EMBED = 768
BATCH = 4
SEQ = 2048
NTOK = BATCH * SEQ

NUM_CORES = 2
NUM_SUBCORES = 16
NUM_WORKERS = NUM_CORES * NUM_SUBCORES
LANES = 16

POS_BLK = SEQ // NUM_WORKERS
ROWS = 16
NBLK = POS_BLK // ROWS
COLS = EMBED // LANES

_mesh = plsc.VectorSubcoreMesh(core_axis_name="c", subcore_axis_name="s")


@functools.partial(
    pl.kernel,
    mesh=_mesh,
    out_type=jax.ShapeDtypeStruct((NTOK, EMBED), jnp.float32),
    scratch_types=[
        pltpu.VMEM((BATCH * POS_BLK,), jnp.int32),
        pltpu.VMEM((BATCH, ROWS, EMBED), jnp.float32),
        pltpu.VMEM((BATCH, ROWS, EMBED), jnp.float32),
        pltpu.VMEM((ROWS, EMBED), jnp.float32),
        pltpu.VMEM((ROWS, EMBED), jnp.float32),
        pltpu.SemaphoreType.DMA,
        pltpu.SemaphoreType.DMA,
        pltpu.SemaphoreType.DMA,
        pltpu.SemaphoreType.DMA,
        pltpu.SemaphoreType.DMA,
    ],
)
def _embed_sc(x_hbm, tok_hbm, pos_hbm, out_hbm,
              idx_v, tokA, tokB, posA, posB,
              isem, gsA, gsB, osA, osB):
    wid = lax.axis_index("s") * NUM_CORES + lax.axis_index("c")
    pbase = wid * POS_BLK

    tbufs = (tokA, tokB)
    pbufs = (posA, posB)
    gsems = (gsA, gsB)
    osems = (osA, osB)

    h_idx = [
        pltpu.async_copy(x_hbm.at[pl.ds(b * SEQ + pbase, POS_BLK)],
                         idx_v.at[pl.ds(b * POS_BLK, POS_BLK)], isem)
        for b in range(BATCH)
    ]

    def issue_step(j):
        cur = j & 1
        hs = [pltpu.async_copy(
            pos_hbm.at[pl.ds(pbase + ROWS * j, ROWS)], pbufs[cur], gsems[cur])]
        for b in range(BATCH):
            hs.append(pltpu.async_copy(
                tok_hbm.at[idx_v.at[pl.ds(b * POS_BLK + ROWS * j, ROWS)]],
                tbufs[cur].at[b], gsems[cur]))
        return hs

    def issue_out(j):
        cur = j & 1
        return [
            pltpu.async_copy(
                tbufs[cur].at[b],
                out_hbm.at[pl.ds(b * SEQ + pbase + ROWS * j, ROWS)],
                osems[cur])
            for b in range(BATCH)
        ]

    for h in h_idx:
        h.wait()
    g = [None] * NBLK
    o = [None] * NBLK
    g[0] = issue_step(0)

    for j in range(NBLK):
        cur = j & 1
        if j + 1 < NBLK:
            if j >= 1:
                for h in o[j - 1]:
                    h.wait()
            g[j + 1] = issue_step(j + 1)
        for h in g[j]:
            h.wait()

        tok = tbufs[cur]
        pos = pbufs[cur]

        def _row(r, carry, tok=tok, pos=pos):
            for c in range(COLS):
                sl = pl.ds(c * LANES, LANES)
                vpos = pos[r, sl]
                for b in range(BATCH):
                    tok[b, r, sl] = tok[b, r, sl] + vpos
            return carry

        lax.fori_loop(0, ROWS, _row, 0)
        o[j] = issue_out(j)

    for h in o[NBLK - 2]:
        h.wait()
    for h in o[NBLK - 1]:
        h.wait()


@jax.jit
def kernel(x, tok_table, pos_table):
    out = _embed_sc(x.reshape(-1), tok_table, pos_table)
    return out.reshape(BATCH, SEQ, EMBED)


# speedup vs baseline: 1.1711x; 1.1711x over previous
import functools

import jax
import jax.numpy as jnp
from jax import lax
from jax.experimental import pallas as pl
from jax.experimental.pallas import tpu as pltpu
from jax.experimental.pallas import tpu_sc as plsc

VOCAB_SIZE = 50257
EMBED = 768
BATCH = 4
SEQ = 2048
NTOK = BATCH * SEQ

NUM_CORES = 2
NUM_SUBCORES = 16
NUM_WORKERS = NUM_CORES * NUM_SUBCORES
LANES = 16

POS_BLK = SEQ // NUM_WORKERS
COLS = EMBED // LANES

_mesh = plsc.VectorSubcoreMesh(core_axis_name="c", subcore_axis_name="s")


@functools.partial(
    pl.kernel,
    mesh=_mesh,
    out_type=jax.ShapeDtypeStruct((NTOK, EMBED), jnp.float32),
    scratch_types=[
        pltpu.VMEM((BATCH * POS_BLK,), jnp.int32),
        pltpu.VMEM((POS_BLK, EMBED), jnp.float32),
        pltpu.VMEM((POS_BLK, EMBED), jnp.float32),
        pltpu.SemaphoreType.DMA,
        pltpu.SemaphoreType.DMA,
        pltpu.SemaphoreType.DMA,
        pltpu.SemaphoreType.DMA,
    ],
)
def _embed_sc(x_hbm, tok_hbm, pos_hbm, out_hbm,
              idx_v, tok_v, pos_v, isem, psem, gsem, osem):
    wid = lax.axis_index("s") * NUM_CORES + lax.axis_index("c")
    pbase = wid * POS_BLK

    h_pos = pltpu.async_copy(pos_hbm.at[pl.ds(pbase, POS_BLK)], pos_v, psem)
    h_idx = [
        pltpu.async_copy(x_hbm.at[pl.ds(b * SEQ + pbase, POS_BLK)],
                         idx_v.at[pl.ds(b * POS_BLK, POS_BLK)], isem)
        for b in range(BATCH)
    ]
    for h in h_idx:
        h.wait()

    h_out = None
    for b in range(BATCH):
        tbase = b * SEQ + pbase
        if h_out is not None:
            h_out.wait()
        h_g = pltpu.async_copy(
            tok_hbm.at[idx_v.at[pl.ds(b * POS_BLK, POS_BLK)]], tok_v, gsem)
        if b == 0:
            h_pos.wait()
        h_g.wait()

        def _row(r, carry):
            for c in range(COLS):
                sl = pl.ds(c * LANES, LANES)
                tok_v[r, sl] = tok_v[r, sl] + pos_v[r, sl]
            return carry

        lax.fori_loop(0, POS_BLK, _row, 0)
        h_out = pltpu.async_copy(tok_v, out_hbm.at[pl.ds(tbase, POS_BLK)],
                                 osem)
    h_out.wait()


@jax.jit
def kernel(x, tok_table, pos_table):
    out = _embed_sc(x.reshape(-1), tok_table, pos_table)
    return out.reshape(BATCH, SEQ, EMBED)


# speedup vs baseline: 1.2038x; 1.0279x over previous
import functools

import jax
import jax.numpy as jnp
from jax import lax
from jax.experimental import pallas as pl
from jax.experimental.pallas import tpu as pltpu
from jax.experimental.pallas import tpu_sc as plsc

VOCAB_SIZE = 50257
EMBED = 768
BATCH = 4
SEQ = 2048
NTOK = BATCH * SEQ

NUM_CORES = 2
NUM_SUBCORES = 16
NUM_WORKERS = NUM_CORES * NUM_SUBCORES
LANES = 16

POS_BLK = SEQ // NUM_WORKERS
COLS = EMBED // LANES

_mesh = plsc.VectorSubcoreMesh(core_axis_name="c", subcore_axis_name="s")


@functools.partial(
    pl.kernel,
    mesh=_mesh,
    out_type=jax.ShapeDtypeStruct((NTOK, EMBED), jnp.float32),
    scratch_types=[
        pltpu.VMEM((BATCH * POS_BLK,), jnp.int32),
        pltpu.VMEM((POS_BLK, EMBED), jnp.float32),
        pltpu.VMEM((POS_BLK, EMBED), jnp.float32),
        pltpu.SemaphoreType.DMA,
        pltpu.SemaphoreType.DMA,
        pltpu.SemaphoreType.DMA,
        pltpu.SemaphoreType.DMA,
    ],
)
def _embed_sc(x_hbm, tok_hbm, pos_hbm, out_hbm,
              idx_v, tok_v, pos_v, isem, psem, gsem, osem):
    wid = lax.axis_index("s") * NUM_CORES + lax.axis_index("c")
    pbase = wid * POS_BLK

    h_pos = pltpu.async_copy(pos_hbm.at[pl.ds(pbase, POS_BLK)], pos_v, psem)
    h_idx = [
        pltpu.async_copy(x_hbm.at[pl.ds(b * SEQ + pbase, POS_BLK)],
                         idx_v.at[pl.ds(b * POS_BLK, POS_BLK)], isem)
        for b in range(BATCH)
    ]
    for h in h_idx:
        h.wait()

    h_pos.wait()

    def _batch(b, carry):
        tbase = b * SEQ + pbase

        @pl.when(b > 0)
        def _():
            pltpu.make_async_copy(
                tok_v, out_hbm.at[pl.ds(tbase, POS_BLK)], osem).wait()

        pltpu.async_copy(
            tok_hbm.at[idx_v.at[pl.ds(b * POS_BLK, POS_BLK)]], tok_v,
            gsem).wait()

        def _row(r, carry2):
            for c in range(COLS):
                sl = pl.ds(c * LANES, LANES)
                tok_v[r, sl] = tok_v[r, sl] + pos_v[r, sl]
            return carry2

        lax.fori_loop(0, POS_BLK, _row, 0)
        pltpu.async_copy(tok_v, out_hbm.at[pl.ds(tbase, POS_BLK)], osem)
        return carry

    lax.fori_loop(0, BATCH, _batch, 0)
    pltpu.make_async_copy(tok_v, out_hbm.at[pl.ds(pbase, POS_BLK)],
                          osem).wait()


@jax.jit
def kernel(x, tok_table, pos_table):
    out = _embed_sc(x.reshape(-1), tok_table, pos_table)
    return out.reshape(BATCH, SEQ, EMBED)
